# baseline (speedup 1.0000x reference)
import functools

import jax
import jax.numpy as jnp
from jax import lax
from jax.experimental import pallas as pl
from jax.experimental.pallas import tpu as pltpu
from jax.experimental.pallas import tpu_sc as plsc

B = 16384
HIST = 20
TOTAL = B * HIST
DIM = 128
NCAT = 1000

NC, NS = 2, 16
NW = NC * NS
BAGS_PW = B // NW
CB = 128
NCHB = BAGS_PW // CB
LANES = 16
DSUB = DIM // LANES


def _seg_sum_sc(deps_r, emb_table):
    mesh = plsc.VectorSubcoreMesh(core_axis_name="c", subcore_axis_name="s")

    @functools.partial(
        pl.kernel,
        out_type=jax.ShapeDtypeStruct((B, DIM), jnp.float32),
        mesh=mesh,
        scratch_types=[
            pltpu.VMEM((NCHB * HIST, CB), jnp.int32),
            pltpu.VMEM((2, CB, DIM), jnp.float32),
            pltpu.SemaphoreType.DMA,
            pltpu.SemaphoreType.DMA,
            pltpu.SemaphoreType.DMA,
            pltpu.SemaphoreType.DMA,
        ],
    )
    def k(deps_hbm, table_hbm, out_hbm, idx_v, acc_v,
          semg0, semg1, semo0, semo1):
        wid = lax.axis_index("s") * NC + lax.axis_index("c")
        semg = (semg0, semg1)
        semo = (semo0, semo1)
        pltpu.sync_copy(deps_hbm.at[wid], idx_v)
        zvec = jnp.zeros((LANES,), jnp.float32)

        def zero(buf):
            for r in range(CB):
                for d in range(DSUB):
                    acc_v[buf, r, pl.ds(d * LANES, LANES)] = zvec

        def fire(c, buf):
            for t in range(HIST):
                pltpu.async_copy(
                    table_hbm.at[idx_v.at[c * HIST + t]], acc_v.at[buf],
                    semg[buf], add=True)

        def drain(c, buf):
            for t in range(HIST):
                pltpu.make_async_copy(
                    table_hbm.at[idx_v.at[c * HIST + t]], acc_v.at[buf],
                    semg[buf]).wait()

        def outstore(c, buf):
            return pltpu.make_async_copy(
                acc_v.at[buf],
                out_hbm.at[pl.ds(wid * BAGS_PW + c * CB, CB)], semo[buf])

        zero(0)
        fire(0, 0)
        zero(1)
        fire(1, 1)
        drain(0, 0)
        outstore(0, 0).start()
        drain(1, 1)
        outstore(1, 1).start()
        outstore(0, 0).wait()
        zero(0)
        fire(2, 0)
        outstore(1, 1).wait()
        zero(1)
        fire(3, 1)
        drain(2, 0)
        outstore(2, 0).start()
        drain(3, 1)
        outstore(3, 1).start()
        outstore(2, 0).wait()
        outstore(3, 1).wait()

    return k(deps_r, emb_table)


def _mm_body(x_ref, w_ref, b_ref, o_ref):
    o_ref[...] = (
        jnp.dot(x_ref[...], w_ref[...], preferred_element_type=jnp.float32)
        + b_ref[...]
    )


def _linear_tc(x, w, b2d):
    BM = 1024
    return pl.pallas_call(
        _mm_body,
        grid=(B // BM,),
        in_specs=[
            pl.BlockSpec((BM, DIM), lambda i: (i, 0)),
            pl.BlockSpec((DIM, NCAT), lambda i: (0, 0)),
            pl.BlockSpec((1, NCAT), lambda i: (0, 0)),
        ],
        out_specs=pl.BlockSpec((BM, NCAT), lambda i: (i, 0)),
        out_shape=jax.ShapeDtypeStruct((B, NCAT), jnp.float32),
    )(x, w, b2d)


def kernel(deps, deps_offsets, emb_table, W_lin, b_lin):
    del deps_offsets
    deps_r = (deps.astype(jnp.int32)
              .reshape(NW, NCHB, CB, HIST)
              .transpose(0, 1, 3, 2)
              .reshape(NW, NCHB * HIST, CB))
    sums = _seg_sum_sc(deps_r, emb_table)
    w = (W_lin.T * (1.0 / HIST)).astype(jnp.float32)
    return _linear_tc(sums, w, b_lin.reshape(1, NCAT))


# speedup vs baseline: 1.0108x; 1.0108x over previous
import functools

import jax
import jax.numpy as jnp
from jax import lax
from jax.experimental import pallas as pl
from jax.experimental.pallas import tpu as pltpu
from jax.experimental.pallas import tpu_sc as plsc

B = 16384
HIST = 20
TOTAL = B * HIST
DIM = 128
NCAT = 1000

NC, NS = 2, 16
NW = NC * NS
CB = 128
LANES = 16
DSUB = DIM // LANES

NHALF = 2
BH = B // NHALF
BAGS_PW = BH // NW
NCHB = BAGS_PW // CB
BM = 1024


def _seg_sum_sc(deps_r, emb_table):
    mesh = plsc.VectorSubcoreMesh(core_axis_name="c", subcore_axis_name="s")

    @functools.partial(
        pl.kernel,
        out_type=jax.ShapeDtypeStruct((BH, DIM), jnp.float32),
        mesh=mesh,
        scratch_types=[
            pltpu.VMEM((NCHB * HIST, CB), jnp.int32),
            pltpu.VMEM((2, CB, DIM), jnp.float32),
            pltpu.SemaphoreType.DMA,
            pltpu.SemaphoreType.DMA,
            pltpu.SemaphoreType.DMA,
            pltpu.SemaphoreType.DMA,
        ],
    )
    def k(deps_hbm, table_hbm, out_hbm, idx_v, acc_v,
          semg0, semg1, semo0, semo1):
        wid = lax.axis_index("s") * NC + lax.axis_index("c")
        semg = (semg0, semg1)
        semo = (semo0, semo1)
        pltpu.sync_copy(deps_hbm.at[wid], idx_v)
        zvec = jnp.zeros((LANES,), jnp.float32)

        def zero(buf):
            for r in range(CB):
                for d in range(DSUB):
                    acc_v[buf, r, pl.ds(d * LANES, LANES)] = zvec

        def fire(c, buf):
            for t in range(HIST):
                pltpu.async_copy(
                    table_hbm.at[idx_v.at[c * HIST + t]], acc_v.at[buf],
                    semg[buf], add=True)

        def drain(c, buf):
            for t in range(HIST):
                pltpu.make_async_copy(
                    table_hbm.at[idx_v.at[c * HIST + t]], acc_v.at[buf],
                    semg[buf]).wait()

        def outstore(c, buf):
            return pltpu.make_async_copy(
                acc_v.at[buf],
                out_hbm.at[pl.ds(wid * BAGS_PW + c * CB, CB)], semo[buf])

        zero(0)
        fire(0, 0)
        zero(1)
        fire(1, 1)
        drain(0, 0)
        outstore(0, 0).start()
        drain(1, 1)
        outstore(1, 1).start()
        outstore(0, 0).wait()
        outstore(1, 1).wait()

    return k(deps_r, emb_table)


def _mm_body(x_ref, w_ref, b_ref, o_ref):
    o_ref[...] = (
        jnp.dot(x_ref[...], w_ref[...], preferred_element_type=jnp.float32)
        + b_ref[...]
    )


def _mm_body_alias(prev_ref, x_ref, w_ref, b_ref, o_ref):
    del prev_ref
    _mm_body(x_ref, w_ref, b_ref, o_ref)


def _linear_tc_first(x, w, b2d):
    return pl.pallas_call(
        _mm_body,
        grid=(BH // BM,),
        in_specs=[
            pl.BlockSpec((BM, DIM), lambda i: (i, 0)),
            pl.BlockSpec((DIM, NCAT), lambda i: (0, 0)),
            pl.BlockSpec((1, NCAT), lambda i: (0, 0)),
        ],
        out_specs=pl.BlockSpec((BM, NCAT), lambda i: (i, 0)),
        out_shape=jax.ShapeDtypeStruct((B, NCAT), jnp.float32),
    )(x, w, b2d)


def _linear_tc_second(out_prev, x, w, b2d):
    nblk = BH // BM
    return pl.pallas_call(
        _mm_body_alias,
        grid=(nblk,),
        in_specs=[
            pl.BlockSpec(memory_space=pltpu.MemorySpace.HBM),
            pl.BlockSpec((BM, DIM), lambda i: (i, 0)),
            pl.BlockSpec((DIM, NCAT), lambda i: (0, 0)),
            pl.BlockSpec((1, NCAT), lambda i: (0, 0)),
        ],
        out_specs=pl.BlockSpec((BM, NCAT), lambda i: (i + nblk, 0)),
        out_shape=jax.ShapeDtypeStruct((B, NCAT), jnp.float32),
        input_output_aliases={0: 0},
    )(out_prev, x, w, b2d)


def kernel(deps, deps_offsets, emb_table, W_lin, b_lin):
    del deps_offsets
    deps_r = (deps.astype(jnp.int32)
              .reshape(NHALF, NW, NCHB, CB, HIST)
              .transpose(0, 1, 2, 4, 3)
              .reshape(NHALF, NW, NCHB * HIST, CB))
    w = (W_lin.T * (1.0 / HIST)).astype(jnp.float32)
    b2d = b_lin.reshape(1, NCAT)
    sums0 = _seg_sum_sc(deps_r[0], emb_table)
    sums1 = _seg_sum_sc(deps_r[1], emb_table)
    out = _linear_tc_first(sums0, w, b2d)
    return _linear_tc_second(out, sums1, w, b2d)


# speedup vs baseline: 1.0317x; 1.0206x over previous
import functools

import jax
import jax.numpy as jnp
from jax import lax
from jax.experimental import pallas as pl
from jax.experimental.pallas import tpu as pltpu
from jax.experimental.pallas import tpu_sc as plsc

B = 16384
HIST = 20
TOTAL = B * HIST
DIM = 128
NCAT = 1000

NC, NS = 2, 16
NW = NC * NS
BAGS_PW = B // NW
CB = 128
NCHB = BAGS_PW // CB
LANES = 16
DSUB = DIM // LANES


def _seg_sum_sc(deps_r, emb_table):
    mesh = plsc.VectorSubcoreMesh(core_axis_name="c", subcore_axis_name="s")

    @functools.partial(
        pl.kernel,
        out_type=jax.ShapeDtypeStruct((B, DIM), jnp.float32),
        mesh=mesh,
        scratch_types=[
            pltpu.VMEM((NCHB * HIST, CB), jnp.int32),
            pltpu.VMEM((2, CB, DIM), jnp.float32),
            pltpu.SemaphoreType.DMA,
            pltpu.SemaphoreType.DMA,
            pltpu.SemaphoreType.DMA,
            pltpu.SemaphoreType.DMA,
        ],
    )
    def k(deps_hbm, table_hbm, out_hbm, idx_v, acc_v,
          semg0, semg1, semo0, semo1):
        wid = lax.axis_index("s") * NC + lax.axis_index("c")
        semg = (semg0, semg1)
        semo = (semo0, semo1)
        pltpu.sync_copy(deps_hbm.at[wid], idx_v)
        zvec = jnp.zeros((LANES,), jnp.float32)

        def zero(buf):
            for r in range(CB):
                for d in range(DSUB):
                    acc_v[buf, r, pl.ds(d * LANES, LANES)] = zvec

        def fire(c, buf):
            for t in range(HIST):
                pltpu.async_copy(
                    table_hbm.at[idx_v.at[c * HIST + t]], acc_v.at[buf],
                    semg[buf], add=True)

        def drain(c, buf):
            for t in range(HIST):
                pltpu.make_async_copy(
                    table_hbm.at[idx_v.at[c * HIST + t]], acc_v.at[buf],
                    semg[buf]).wait()

        def outstore(c, buf):
            return pltpu.make_async_copy(
                acc_v.at[buf],
                out_hbm.at[pl.ds(wid * BAGS_PW + c * CB, CB)], semo[buf])

        def pair_body(i, carry):
            c0 = 2 * i
            zero(0)
            fire(c0, 0)
            zero(1)
            fire(c0 + 1, 1)
            drain(c0, 0)
            outstore(c0, 0).start()
            drain(c0 + 1, 1)
            outstore(c0 + 1, 1).start()
            outstore(c0, 0).wait()
            outstore(c0 + 1, 1).wait()
            return carry

        lax.fori_loop(0, NCHB // 2, pair_body, 0)

    return k(deps_r, emb_table)


def _mm_body(x_ref, w_ref, b_ref, o_ref):
    x = x_ref[...] * (1.0 / HIST)
    o_ref[...] = (
        lax.dot_general(x, w_ref[...], (((1,), (1,)), ((), ())),
                        preferred_element_type=jnp.float32)
        + b_ref[...]
    )


def _linear_tc(x, w, b2d):
    BM = 2048
    return pl.pallas_call(
        _mm_body,
        grid=(B // BM,),
        in_specs=[
            pl.BlockSpec((BM, DIM), lambda i: (i, 0)),
            pl.BlockSpec((NCAT, DIM), lambda i: (0, 0)),
            pl.BlockSpec((1, NCAT), lambda i: (0, 0)),
        ],
        out_specs=pl.BlockSpec((BM, NCAT), lambda i: (i, 0)),
        out_shape=jax.ShapeDtypeStruct((B, NCAT), jnp.float32),
    )(x, w, b2d)


def kernel(deps, deps_offsets, emb_table, W_lin, b_lin):
    del deps_offsets
    deps_r = (deps.astype(jnp.int32)
              .reshape(NW, NCHB, CB, HIST)
              .transpose(0, 1, 3, 2)
              .reshape(NW, NCHB * HIST, CB))
    sums = _seg_sum_sc(deps_r, emb_table)
    return _linear_tc(sums, W_lin, b_lin.reshape(1, NCAT))
